# baseline (speedup 1.0000x reference)
import functools

import jax
import jax.numpy as jnp
from jax import lax
from jax.experimental import pallas as pl
from jax.experimental.pallas import tpu as pltpu
from jax.experimental.pallas import tpu_sc as plsc

V = 9217
D = 192
DA = 128
DB = D - DA
NC = 2
NS = 16
NW = NC * NS
CHUNK = 64
NBUF = 2
ROWS_PER_SUB = V // NS


@functools.partial(jax.jit, static_argnames=("batch", "seq"))
def _lookup(ids_flat, table, *, batch, seq):
    B = batch * seq
    per_w = B // NW
    G = per_w // CHUNK

    ids3 = ids_flat.reshape(NW, G // 2, 2 * CHUNK)
    table_a = table[:, :DA]
    table_b = jnp.pad(table[:, DA:], ((0, 0), (0, DA - DB)))

    mesh = plsc.VectorSubcoreMesh(core_axis_name="c", subcore_axis_name="s")

    @functools.partial(
        pl.kernel,
        out_type=jax.ShapeDtypeStruct((B, D), jnp.float32),
        mesh=mesh,
        scratch_types=[
            pltpu.VMEM_SHARED((V, DA), jnp.float32),
            pltpu.VMEM((G // 2, 2 * CHUNK), jnp.int32),
            [pltpu.VMEM((CHUNK, DA), jnp.float32) for _ in range(NBUF)],
            [pltpu.VMEM((CHUNK, DA), jnp.float32) for _ in range(NBUF)],
            [pltpu.VMEM((CHUNK, DB), jnp.float32) for _ in range(NBUF)],
            [pltpu.SemaphoreType.DMA for _ in range(NBUF)],
            [pltpu.SemaphoreType.DMA for _ in range(NBUF)],
            [pltpu.SemaphoreType.DMA for _ in range(NBUF)],
            [pltpu.SemaphoreType.DMA for _ in range(NBUF)],
        ],
    )
    def k(ids_hbm, ta_hbm, tb_hbm, out_hbm, shared_a, idx_v, bufa, bufb,
          bufb64, gsa, gsb, wsa, wsb):
        sid = lax.axis_index("s")
        wid = sid * NC + lax.axis_index("c")
        base = wid * per_w

        pltpu.sync_copy(
            ta_hbm.at[pl.ds(sid * ROWS_PER_SUB, ROWS_PER_SUB)],
            shared_a.at[pl.ds(sid * ROWS_PER_SUB, ROWS_PER_SUB)],
        )

        @pl.when(sid == 0)
        def _():
            pltpu.sync_copy(
                ta_hbm.at[pl.ds(NS * ROWS_PER_SUB, V - NS * ROWS_PER_SUB)],
                shared_a.at[pl.ds(NS * ROWS_PER_SUB, V - NS * ROWS_PER_SUB)],
            )

        pltpu.sync_copy(ids_hbm.at[wid], idx_v)
        plsc.subcore_barrier()

        def idx_ref(g, parity):
            return idx_v.at[g // 2, pl.ds(parity * CHUNK, CHUNK)]

        def start_gather(g, parity, b):
            ix = idx_ref(g, parity)
            pltpu.async_copy(shared_a.at[ix], bufa[b], gsa[b])
            pltpu.async_copy(tb_hbm.at[ix], bufb[b], gsb[b])

        def wait_gather(g, parity, b):
            ix = idx_ref(g, parity)
            pltpu.make_async_copy(shared_a.at[ix], bufa[b], gsa[b]).wait()
            pltpu.make_async_copy(tb_hbm.at[ix], bufb[b], gsb[b]).wait()

        def compact_b(b):
            @pl.loop(0, CHUNK, unroll=4)
            def _(r):
                for c in range(DB // 16):
                    bufb64[b][r, pl.ds(c * 16, 16)] = (
                        bufb[b][r, pl.ds(c * 16, 16)]
                    )

        def start_write(g, b):
            r0 = base + g * CHUNK
            pltpu.async_copy(
                bufa[b], out_hbm.at[pl.ds(r0, CHUNK), pl.ds(0, DA)], wsa[b]
            )
            pltpu.async_copy(
                bufb64[b],
                out_hbm.at[pl.ds(r0, CHUNK), pl.ds(DA, DB)],
                wsb[b],
            )

        def wait_write(g, b):
            r0 = base + g * CHUNK
            pltpu.make_async_copy(
                bufa[b], out_hbm.at[pl.ds(r0, CHUNK), pl.ds(0, DA)], wsa[b]
            ).wait()
            pltpu.make_async_copy(
                bufb64[b],
                out_hbm.at[pl.ds(r0, CHUNK), pl.ds(DA, DB)],
                wsb[b],
            ).wait()

        start_gather(0, 0, 0)

        @pl.loop(0, G, step=NBUF)
        def _(g0):
            for j in range(NBUF):
                g = g0 + j
                b = j
                wait_gather(g, j % 2, b)
                compact_b(b)
                start_write(g, b)
                b2 = (j + 1) % NBUF

                @pl.when(g >= 1)
                def _():
                    wait_write(g - 1, b2)

                @pl.when(g + 1 < G)
                def _():
                    start_gather(g + 1, (j + 1) % 2, b2)

        wait_write(G - 1, (G - 1) % NBUF)

    return k(ids3, table_a, table_b)


def kernel(position_ids, table):
    batch, seq = position_ids.shape
    ids_flat = position_ids.reshape(-1).astype(jnp.int32)
    out = _lookup(ids_flat, table, batch=batch, seq=seq)
    return out.reshape(batch, seq, D)


# speedup vs baseline: 1.0495x; 1.0495x over previous
import functools

import jax
import jax.numpy as jnp
from jax import lax
from jax.experimental import pallas as pl
from jax.experimental.pallas import tpu as pltpu
from jax.experimental.pallas import tpu_sc as plsc

V = 9217
D = 192
DA = 128
DB = D - DA
NC = 2
NS = 16
NW = NC * NS
CHUNK = 128
NBA = 3
NBB = 2
STEP = 6


@functools.partial(jax.jit, static_argnames=("batch", "seq"))
def _lookup(ids_flat, table, *, batch, seq):
    B = batch * seq
    per_w = B // NW
    G = per_w // CHUNK

    ids3 = ids_flat.reshape(NW, G, CHUNK)
    table_a = table[:, :DA]
    table_b = jnp.pad(table[:, DA:], ((0, 0), (0, DA - DB)))

    mesh = plsc.VectorSubcoreMesh(core_axis_name="c", subcore_axis_name="s")

    @functools.partial(
        pl.kernel,
        out_type=jax.ShapeDtypeStruct((B, D), jnp.float32),
        mesh=mesh,
        scratch_types=[
            pltpu.VMEM((G, CHUNK), jnp.int32),
            [pltpu.VMEM((CHUNK, DA), jnp.float32) for _ in range(NBA)],
            [pltpu.VMEM((CHUNK, DA), jnp.float32) for _ in range(NBB)],
            [pltpu.VMEM((CHUNK, DB), jnp.float32) for _ in range(NBB)],
            [pltpu.SemaphoreType.DMA for _ in range(NBA)],
            [pltpu.SemaphoreType.DMA for _ in range(NBB)],
            [pltpu.SemaphoreType.DMA for _ in range(NBA)],
            [pltpu.SemaphoreType.DMA for _ in range(NBB)],
        ],
    )
    def k(ids_hbm, ta_hbm, tb_hbm, out_hbm, idx_v, bufa, bufb, bufb64,
          gsa, gsb, wsa, wsb):
        wid = lax.axis_index("s") * NC + lax.axis_index("c")
        base = wid * per_w

        pltpu.sync_copy(ids_hbm.at[wid], idx_v)

        def start_gather_a(g, ba):
            pltpu.async_copy(ta_hbm.at[idx_v.at[g]], bufa[ba], gsa[ba])

        def wait_gather_a(g, ba):
            pltpu.make_async_copy(
                ta_hbm.at[idx_v.at[g]], bufa[ba], gsa[ba]
            ).wait()

        def start_gather_b(g, bb):
            pltpu.async_copy(tb_hbm.at[idx_v.at[g]], bufb[bb], gsb[bb])

        def wait_gather_b(g, bb):
            pltpu.make_async_copy(
                tb_hbm.at[idx_v.at[g]], bufb[bb], gsb[bb]
            ).wait()

        def compact_b(b):
            @pl.loop(0, CHUNK, unroll=4)
            def _(r):
                for c in range(DB // 16):
                    bufb64[b][r, pl.ds(c * 16, 16)] = (
                        bufb[b][r, pl.ds(c * 16, 16)]
                    )

        def start_write(g, ba, bb):
            r0 = base + g * CHUNK
            pltpu.async_copy(
                bufa[ba], out_hbm.at[pl.ds(r0, CHUNK), pl.ds(0, DA)], wsa[ba]
            )
            pltpu.async_copy(
                bufb64[bb],
                out_hbm.at[pl.ds(r0, CHUNK), pl.ds(DA, DB)],
                wsb[bb],
            )

        def wait_write(g, ba, bb):
            r0 = base + g * CHUNK
            pltpu.make_async_copy(
                bufa[ba], out_hbm.at[pl.ds(r0, CHUNK), pl.ds(0, DA)], wsa[ba]
            ).wait()
            pltpu.make_async_copy(
                bufb64[bb],
                out_hbm.at[pl.ds(r0, CHUNK), pl.ds(DA, DB)],
                wsb[bb],
            ).wait()

        start_gather_a(0, 0)
        start_gather_a(1, 1)
        start_gather_b(0, 0)

        @pl.loop(0, G, step=STEP)
        def _(g0):
            for j in range(STEP):
                g = g0 + j
                ba = j % NBA
                bb = j % NBB
                wait_gather_a(g, ba)
                wait_gather_b(g, bb)

                @pl.when(g >= 1)
                def _():
                    wait_write(g - 1, (j + 2) % NBA, (j + 1) % NBB)

                compact_b(bb)
                start_write(g, ba, bb)

                @pl.when(g + 2 < G)
                def _():
                    start_gather_a(g + 2, (j + 2) % NBA)

                @pl.when(g + 1 < G)
                def _():
                    start_gather_b(g + 1, (j + 1) % NBB)

        wait_write(G - 1, (G - 1) % NBA, (G - 1) % NBB)

    return k(ids3, table_a, table_b)


def kernel(position_ids, table):
    batch, seq = position_ids.shape
    ids_flat = position_ids.reshape(-1).astype(jnp.int32)
    out = _lookup(ids_flat, table, batch=batch, seq=seq)
    return out.reshape(batch, seq, D)


# speedup vs baseline: 1.2277x; 1.1697x over previous
import functools

import jax
import jax.numpy as jnp
from jax import lax
from jax.experimental import pallas as pl
from jax.experimental.pallas import tpu as pltpu
from jax.experimental.pallas import tpu_sc as plsc

V = 9217
D = 192
DA = 128
DB = D - DA
NC = 2
NS = 16
NW = NC * NS
CHUNK = 128
NBUF = 2


@functools.partial(jax.jit, static_argnames=("batch", "seq"))
def _lookup(ids_flat, table, *, batch, seq):
    B = batch * seq
    per_w = B // NW
    G = per_w // CHUNK

    ids3 = ids_flat.reshape(NW, G, CHUNK)
    table_a = table[:, :DA]
    table_b = jnp.pad(table[:, DA:], ((0, 0), (0, DA - DB)))

    mesh = plsc.VectorSubcoreMesh(core_axis_name="c", subcore_axis_name="s")

    @functools.partial(
        pl.kernel,
        out_type=jax.ShapeDtypeStruct((B, D), jnp.float32),
        mesh=mesh,
        scratch_types=[
            pltpu.VMEM((G, CHUNK), jnp.int32),
            [pltpu.VMEM((CHUNK, DA), jnp.float32) for _ in range(NBUF)],
            [pltpu.VMEM((CHUNK, DA), jnp.float32) for _ in range(NBUF)],
            [pltpu.VMEM((CHUNK, DB), jnp.float32) for _ in range(NBUF)],
            [pltpu.SemaphoreType.DMA for _ in range(NBUF)],
            [pltpu.SemaphoreType.DMA for _ in range(NBUF)],
            [pltpu.SemaphoreType.DMA for _ in range(NBUF)],
            [pltpu.SemaphoreType.DMA for _ in range(NBUF)],
        ],
    )
    def k(ids_hbm, ta_hbm, tb_hbm, out_hbm, idx_v, bufa, bufb, bufb64,
          gsa, gsb, wsa, wsb):
        wid = lax.axis_index("s") * NC + lax.axis_index("c")
        base = wid * per_w

        pltpu.sync_copy(ids_hbm.at[wid], idx_v)

        def start_gather(g, b):
            pltpu.async_copy(ta_hbm.at[idx_v.at[g]], bufa[b], gsa[b])
            pltpu.async_copy(tb_hbm.at[idx_v.at[g]], bufb[b], gsb[b])

        def wait_gather(g, b):
            pltpu.make_async_copy(ta_hbm.at[idx_v.at[g]], bufa[b], gsa[b]).wait()
            pltpu.make_async_copy(tb_hbm.at[idx_v.at[g]], bufb[b], gsb[b]).wait()

        def compact_b(b):
            @pl.loop(0, CHUNK)
            def _(r):
                for c in range(DB // 16):
                    bufb64[b][r, pl.ds(c * 16, 16)] = (
                        bufb[b][r, pl.ds(c * 16, 16)]
                    )

        def start_write(g, b):
            r0 = base + g * CHUNK
            pltpu.async_copy(
                bufa[b], out_hbm.at[pl.ds(r0, CHUNK), pl.ds(0, DA)], wsa[b]
            )
            pltpu.async_copy(
                bufb64[b],
                out_hbm.at[pl.ds(r0, CHUNK), pl.ds(DA, DB)],
                wsb[b],
            )

        def wait_write(g, b):
            r0 = base + g * CHUNK
            pltpu.make_async_copy(
                bufa[b], out_hbm.at[pl.ds(r0, CHUNK), pl.ds(0, DA)], wsa[b]
            ).wait()
            pltpu.make_async_copy(
                bufb64[b],
                out_hbm.at[pl.ds(r0, CHUNK), pl.ds(DA, DB)],
                wsb[b],
            ).wait()

        start_gather(0, 0)

        @pl.loop(0, G, step=NBUF)
        def _(g0):
            for j in range(NBUF):
                g = g0 + j
                b = j
                wait_gather(g, b)
                b2 = (j + 1) % NBUF

                @pl.when(g >= 1)
                def _():
                    wait_write(g - 1, b2)

                @pl.when(g + 1 < G)
                def _():
                    start_gather(g + 1, b2)

                compact_b(b)
                start_write(g, b)

        wait_write(G - 1, (G - 1) % NBUF)

    return k(ids3, table_a, table_b)


def kernel(position_ids, table):
    batch, seq = position_ids.shape
    ids_flat = position_ids.reshape(-1).astype(jnp.int32)
    out = _lookup(ids_flat, table, batch=batch, seq=seq)
    return out.reshape(batch, seq, D)
